# Initial kernel scaffold; baseline (speedup 1.0000x reference)
import jax
import jax.numpy as jnp
from jax.experimental import pallas as pl


def kernel(pos, edge_index, edge_weight):
    raise NotImplementedError("write your pallas kernel here")


# speedup vs baseline: 6.2551x; 6.2551x over previous
import functools

import jax
import jax.numpy as jnp
from jax import lax
from jax.experimental import pallas as pl
from jax.experimental.pallas import tpu as pltpu
from jax.experimental.pallas import tpu_sc as plsc

NC = 2
NS = 16
NW = NC * NS
L = 16

C = 2048


def _worker_id():
    return lax.axis_index("s") * NC + lax.axis_index("c")


def _num_chunks(wid, total_chunks):
    return (total_chunks - wid + NW - 1) // NW


def _max_body(row_hbm, col_hbm, px_hbm, py_hbm, pz_hbm, maxes_hbm,
              ridx, cidx, xr, yr, zr, xc, yc, zc, mbuf, sem):
    total_chunks = row_hbm.shape[0] // C
    wid = _worker_id()
    nchunks = _num_chunks(wid, total_chunks)

    def chunk_body(j, m):
        base = (wid + j * NW) * C
        pltpu.sync_copy(row_hbm.at[pl.ds(base, C)], ridx)
        pltpu.sync_copy(col_hbm.at[pl.ds(base, C)], cidx)
        cps = [
            pltpu.async_copy(px_hbm.at[ridx], xr, sem),
            pltpu.async_copy(py_hbm.at[ridx], yr, sem),
            pltpu.async_copy(pz_hbm.at[ridx], zr, sem),
            pltpu.async_copy(px_hbm.at[cidx], xc, sem),
            pltpu.async_copy(py_hbm.at[cidx], yc, sem),
            pltpu.async_copy(pz_hbm.at[cidx], zc, sem),
        ]
        for cp in cps:
            cp.wait()

        def vbody(k, m):
            s16 = pl.ds(k * L, L)
            dx = jnp.abs(xc[s16] - xr[s16])
            dy = jnp.abs(yc[s16] - yr[s16])
            dz = jnp.abs(zc[s16] - zr[s16])
            return jnp.maximum(jnp.maximum(m, dx), jnp.maximum(dy, dz))

        return lax.fori_loop(0, C // L, vbody, m)

    m = lax.fori_loop(0, nchunks, chunk_body, jnp.zeros((L,), jnp.float32))
    mbuf[...] = m
    pltpu.sync_copy(mbuf, maxes_hbm.at[wid])


def _out_body(row_hbm, col_hbm, ew_hbm, px_hbm, py_hbm, pz_hbm, maxes_hbm,
              out_hbm,
              ridx, cidx, xr, yr, zr, xc, yc, zc, ewb, ob, mvb, sem):
    total_chunks = row_hbm.shape[0] // C
    wid = _worker_id()
    nchunks = _num_chunks(wid, total_chunks)

    pltpu.sync_copy(maxes_hbm, mvb)

    def mred(i, m):
        return jnp.maximum(m, mvb[i, :])

    m16 = lax.fori_loop(0, NW, mred, jnp.zeros((L,), jnp.float32))
    iota = lax.broadcasted_iota(jnp.int32, (L,), 0)
    for sh in (8, 4, 2, 1):
        perm = jnp.bitwise_and(iota + sh, L - 1)
        m16 = jnp.maximum(m16, m16.at[perm].get(mode="promise_in_bounds"))
    sv = 1.0 / (2.0 * m16)
    idx_x = iota * 4

    def chunk_body(j, _):
        base = (wid + j * NW) * C
        pltpu.sync_copy(row_hbm.at[pl.ds(base, C)], ridx)
        pltpu.sync_copy(col_hbm.at[pl.ds(base, C)], cidx)
        cps = [
            pltpu.async_copy(px_hbm.at[ridx], xr, sem),
            pltpu.async_copy(py_hbm.at[ridx], yr, sem),
            pltpu.async_copy(pz_hbm.at[ridx], zr, sem),
            pltpu.async_copy(px_hbm.at[cidx], xc, sem),
            pltpu.async_copy(py_hbm.at[cidx], yc, sem),
            pltpu.async_copy(pz_hbm.at[cidx], zc, sem),
        ]
        pltpu.sync_copy(ew_hbm.at[pl.ds(base, C)], ewb)
        for cp in cps:
            cp.wait()

        def vbody(k, _):
            s16 = pl.ds(k * L, L)
            dx = (xc[s16] - xr[s16]) * sv + 0.5
            dy = (yc[s16] - yr[s16]) * sv + 0.5
            dz = (zc[s16] - zr[s16]) * sv + 0.5
            ewv = ewb[s16]
            b4 = idx_x + k * (4 * L)
            plsc.store_scatter(ob, [b4], dx)
            plsc.store_scatter(ob, [b4 + 1], dy)
            plsc.store_scatter(ob, [b4 + 2], dz)
            plsc.store_scatter(ob, [b4 + 3], ewv)
            return 0

        lax.fori_loop(0, C // L, vbody, 0)
        pltpu.sync_copy(ob, out_hbm.at[pl.ds(4 * base, 4 * C)])
        return 0

    lax.fori_loop(0, nchunks, chunk_body, 0)


def kernel(pos, edge_index, edge_weight):
    n = pos.shape[0]
    e = edge_weight.shape[0]
    assert e % C == 0

    row = edge_index[0].astype(jnp.int32)
    col = edge_index[1].astype(jnp.int32)
    px = pos[:, 0]
    py = pos[:, 1]
    pz = pos[:, 2]

    mesh = plsc.VectorSubcoreMesh(core_axis_name="c", subcore_axis_name="s")

    cparams = pltpu.CompilerParams(needs_layout_passes=False)

    max_k = pl.kernel(
        _max_body,
        out_type=jax.ShapeDtypeStruct((NW, L), jnp.float32),
        mesh=mesh,
        compiler_params=cparams,
        scratch_types=[
            pltpu.VMEM((C,), jnp.int32),
            pltpu.VMEM((C,), jnp.int32),
            pltpu.VMEM((C,), jnp.float32),
            pltpu.VMEM((C,), jnp.float32),
            pltpu.VMEM((C,), jnp.float32),
            pltpu.VMEM((C,), jnp.float32),
            pltpu.VMEM((C,), jnp.float32),
            pltpu.VMEM((C,), jnp.float32),
            pltpu.VMEM((L,), jnp.float32),
            pltpu.SemaphoreType.DMA,
        ],
    )
    maxes = max_k(row, col, px, py, pz)

    out_k = pl.kernel(
        _out_body,
        out_type=jax.ShapeDtypeStruct((4 * e,), jnp.float32),
        mesh=mesh,
        compiler_params=cparams,
        scratch_types=[
            pltpu.VMEM((C,), jnp.int32),
            pltpu.VMEM((C,), jnp.int32),
            pltpu.VMEM((C,), jnp.float32),
            pltpu.VMEM((C,), jnp.float32),
            pltpu.VMEM((C,), jnp.float32),
            pltpu.VMEM((C,), jnp.float32),
            pltpu.VMEM((C,), jnp.float32),
            pltpu.VMEM((C,), jnp.float32),
            pltpu.VMEM((C,), jnp.float32),
            pltpu.VMEM((4 * C,), jnp.float32),
            pltpu.VMEM((NW, L), jnp.float32),
            pltpu.SemaphoreType.DMA,
        ],
    )
    out = out_k(row, col, edge_weight, px, py, pz, maxes)
    return out.reshape(e, 4)


# speedup vs baseline: 8.7326x; 1.3961x over previous
import functools

import jax
import jax.numpy as jnp
from jax import lax
from jax.experimental import pallas as pl
from jax.experimental.pallas import tpu as pltpu
from jax.experimental.pallas import tpu_sc as plsc

NC = 2
NS = 16
NW = NC * NS
L = 16

C = 2048


def _worker_id():
    return lax.axis_index("s") * NC + lax.axis_index("c")


def _num_chunks(wid, total_chunks):
    return (total_chunks - wid + NW - 1) // NW


def _stage_pos(px_hbm, py_hbm, pz_hbm, px_sh, py_sh, pz_sh):
    @pl.when(lax.axis_index("s") == 0)
    def _():
        pltpu.sync_copy(px_hbm, px_sh)
        pltpu.sync_copy(py_hbm, py_sh)
        pltpu.sync_copy(pz_hbm, pz_sh)

    plsc.subcore_barrier()


def _max_body(row_hbm, col_hbm, px_hbm, py_hbm, pz_hbm, maxes_hbm,
              ridx, cidx, xr, yr, zr, xc, yc, zc, mbuf,
              px_sh, py_sh, pz_sh, sem):
    total_chunks = row_hbm.shape[0] // C
    wid = _worker_id()
    nchunks = _num_chunks(wid, total_chunks)
    _stage_pos(px_hbm, py_hbm, pz_hbm, px_sh, py_sh, pz_sh)

    def chunk_body(j, m):
        base = (wid + j * NW) * C
        pltpu.sync_copy(row_hbm.at[pl.ds(base, C)], ridx)
        pltpu.sync_copy(col_hbm.at[pl.ds(base, C)], cidx)
        cps = [
            pltpu.async_copy(px_sh.at[ridx], xr, sem),
            pltpu.async_copy(py_sh.at[ridx], yr, sem),
            pltpu.async_copy(pz_sh.at[ridx], zr, sem),
            pltpu.async_copy(px_sh.at[cidx], xc, sem),
            pltpu.async_copy(py_sh.at[cidx], yc, sem),
            pltpu.async_copy(pz_sh.at[cidx], zc, sem),
        ]
        for cp in cps:
            cp.wait()

        def vbody(k, m):
            s16 = pl.ds(k * L, L)
            dx = jnp.abs(xc[s16] - xr[s16])
            dy = jnp.abs(yc[s16] - yr[s16])
            dz = jnp.abs(zc[s16] - zr[s16])
            return jnp.maximum(jnp.maximum(m, dx), jnp.maximum(dy, dz))

        return lax.fori_loop(0, C // L, vbody, m)

    m = lax.fori_loop(0, nchunks, chunk_body, jnp.zeros((L,), jnp.float32))
    mbuf[...] = m
    pltpu.sync_copy(mbuf, maxes_hbm.at[wid])


def _out_body(row_hbm, col_hbm, ew_hbm, px_hbm, py_hbm, pz_hbm, maxes_hbm,
              out_hbm,
              ridx, cidx, xr, yr, zr, xc, yc, zc, ewb, ob, mvb,
              px_sh, py_sh, pz_sh, sem):
    total_chunks = row_hbm.shape[0] // C
    wid = _worker_id()
    nchunks = _num_chunks(wid, total_chunks)
    _stage_pos(px_hbm, py_hbm, pz_hbm, px_sh, py_sh, pz_sh)

    pltpu.sync_copy(maxes_hbm, mvb)

    def mred(i, m):
        return jnp.maximum(m, mvb[i, :])

    m16 = lax.fori_loop(0, NW, mred, jnp.zeros((L,), jnp.float32))
    iota = lax.broadcasted_iota(jnp.int32, (L,), 0)
    for sh in (8, 4, 2, 1):
        perm = jnp.bitwise_and(iota + sh, L - 1)
        m16 = jnp.maximum(m16, m16.at[perm].get(mode="promise_in_bounds"))
    sv = 1.0 / (2.0 * m16)
    idx_x = iota * 4

    def chunk_body(j, _):
        base = (wid + j * NW) * C
        pltpu.sync_copy(row_hbm.at[pl.ds(base, C)], ridx)
        pltpu.sync_copy(col_hbm.at[pl.ds(base, C)], cidx)
        cps = [
            pltpu.async_copy(px_sh.at[ridx], xr, sem),
            pltpu.async_copy(py_sh.at[ridx], yr, sem),
            pltpu.async_copy(pz_sh.at[ridx], zr, sem),
            pltpu.async_copy(px_sh.at[cidx], xc, sem),
            pltpu.async_copy(py_sh.at[cidx], yc, sem),
            pltpu.async_copy(pz_sh.at[cidx], zc, sem),
        ]
        pltpu.sync_copy(ew_hbm.at[pl.ds(base, C)], ewb)
        for cp in cps:
            cp.wait()

        def vbody(k, _):
            s16 = pl.ds(k * L, L)
            dx = (xc[s16] - xr[s16]) * sv + 0.5
            dy = (yc[s16] - yr[s16]) * sv + 0.5
            dz = (zc[s16] - zr[s16]) * sv + 0.5
            ewv = ewb[s16]
            b4 = idx_x + k * (4 * L)
            plsc.store_scatter(ob, [b4], dx)
            plsc.store_scatter(ob, [b4 + 1], dy)
            plsc.store_scatter(ob, [b4 + 2], dz)
            plsc.store_scatter(ob, [b4 + 3], ewv)
            return 0

        lax.fori_loop(0, C // L, vbody, 0)
        pltpu.sync_copy(ob, out_hbm.at[pl.ds(4 * base, 4 * C)])
        return 0

    lax.fori_loop(0, nchunks, chunk_body, 0)


def kernel(pos, edge_index, edge_weight):
    n = pos.shape[0]
    e = edge_weight.shape[0]
    assert e % C == 0

    row = edge_index[0].astype(jnp.int32)
    col = edge_index[1].astype(jnp.int32)
    px = pos[:, 0]
    py = pos[:, 1]
    pz = pos[:, 2]

    mesh = plsc.VectorSubcoreMesh(core_axis_name="c", subcore_axis_name="s")

    cparams = pltpu.CompilerParams(needs_layout_passes=False)

    max_k = pl.kernel(
        _max_body,
        out_type=jax.ShapeDtypeStruct((NW, L), jnp.float32),
        mesh=mesh,
        compiler_params=cparams,
        scratch_types=[
            pltpu.VMEM((C,), jnp.int32),
            pltpu.VMEM((C,), jnp.int32),
            pltpu.VMEM((C,), jnp.float32),
            pltpu.VMEM((C,), jnp.float32),
            pltpu.VMEM((C,), jnp.float32),
            pltpu.VMEM((C,), jnp.float32),
            pltpu.VMEM((C,), jnp.float32),
            pltpu.VMEM((C,), jnp.float32),
            pltpu.VMEM((L,), jnp.float32),
            pltpu.VMEM_SHARED((n,), jnp.float32),
            pltpu.VMEM_SHARED((n,), jnp.float32),
            pltpu.VMEM_SHARED((n,), jnp.float32),
            pltpu.SemaphoreType.DMA,
        ],
    )
    maxes = max_k(row, col, px, py, pz)

    out_k = pl.kernel(
        _out_body,
        out_type=jax.ShapeDtypeStruct((4 * e,), jnp.float32),
        mesh=mesh,
        compiler_params=cparams,
        scratch_types=[
            pltpu.VMEM((C,), jnp.int32),
            pltpu.VMEM((C,), jnp.int32),
            pltpu.VMEM((C,), jnp.float32),
            pltpu.VMEM((C,), jnp.float32),
            pltpu.VMEM((C,), jnp.float32),
            pltpu.VMEM((C,), jnp.float32),
            pltpu.VMEM((C,), jnp.float32),
            pltpu.VMEM((C,), jnp.float32),
            pltpu.VMEM((C,), jnp.float32),
            pltpu.VMEM((4 * C,), jnp.float32),
            pltpu.VMEM((NW, L), jnp.float32),
            pltpu.VMEM_SHARED((n,), jnp.float32),
            pltpu.VMEM_SHARED((n,), jnp.float32),
            pltpu.VMEM_SHARED((n,), jnp.float32),
            pltpu.SemaphoreType.DMA,
        ],
    )
    out = out_k(row, col, edge_weight, px, py, pz, maxes)
    return out.reshape(e, 4)
